# speedup vs baseline: 3.0451x; 3.0451x over previous
import functools

import jax
import jax.numpy as jnp
from jax import lax
from jax.experimental import pallas as pl
from jax.experimental.pallas import tpu as pltpu
from jax.experimental.pallas import tpu_sc as plsc

NODE_NUM = 100000
EMBED_DIM = 128
BATCH = 16384
HIST = 50

NC = 2
NS = 16
NW = NC * NS

TOTAL = BATCH * HIST
PER_W = TOTAL // NW
CHUNK = 128
NCHUNK = PER_W // CHUNK


def _body(idx_hbm, w_hbm, out_hbm, idx_v, buf_v, gsem):
    wid = lax.axis_index("s") * NC + lax.axis_index("c")
    pltpu.sync_copy(idx_hbm.at[wid], idx_v)
    base = wid * PER_W

    def step(j, carry):
        pltpu.make_async_copy(w_hbm.at[idx_v.at[j]], buf_v, gsem).start()
        pltpu.make_async_copy(w_hbm.at[idx_v.at[j]], buf_v, gsem).wait()
        pltpu.sync_copy(buf_v, out_hbm.at[pl.ds(base + j * CHUNK, CHUNK)])
        return carry

    lax.fori_loop(0, NCHUNK, step, 0)


@jax.jit
def _run(targ_flat, W):
    idx = targ_flat.reshape(NW, NCHUNK, CHUNK)
    mesh = plsc.VectorSubcoreMesh(core_axis_name="c", subcore_axis_name="s")
    k = pl.kernel(
        _body,
        out_type=jax.ShapeDtypeStruct((TOTAL, EMBED_DIM), jnp.float32),
        mesh=mesh,
        scratch_types=[
            pltpu.VMEM((NCHUNK, CHUNK), jnp.int32),
            pltpu.VMEM((CHUNK, EMBED_DIM), jnp.float32),
            pltpu.SemaphoreType.DMA,
        ],
    )
    return k(idx, W)


def kernel(targ, W):
    out = _run(targ.reshape(TOTAL).astype(jnp.int32), W)
    return out.reshape(BATCH, HIST, EMBED_DIM)


# speedup vs baseline: 3.4472x; 1.1321x over previous
import functools

import jax
import jax.numpy as jnp
from jax import lax
from jax.experimental import pallas as pl
from jax.experimental.pallas import tpu as pltpu
from jax.experimental.pallas import tpu_sc as plsc

NODE_NUM = 100000
EMBED_DIM = 128
BATCH = 16384
HIST = 50

NC = 2
NS = 16
NW = NC * NS

TOTAL = BATCH * HIST
PER_W = TOTAL // NW
CHUNK = 128
NCHUNK = PER_W // CHUNK


NBUF = 4
PREF = 2


def _body(idx_hbm, w_hbm, out_hbm, idx_v, buf_v, gsems, wsems):
    wid = lax.axis_index("s") * NC + lax.axis_index("c")
    pltpu.sync_copy(idx_hbm.at[wid], idx_v)
    base = wid * PER_W

    def gather(g, bg):
        pltpu.make_async_copy(
            w_hbm.at[idx_v.at[g]], buf_v.at[bg], gsems.at[bg]
        ).start()

    for b in range(PREF):
        gather(b, b)

    def step(j, carry):
        b = lax.rem(j, NBUF)
        pltpu.make_async_copy(
            w_hbm.at[idx_v.at[j]], buf_v.at[b], gsems.at[b]
        ).wait()
        pltpu.make_async_copy(
            buf_v.at[b], out_hbm.at[pl.ds(base + j * CHUNK, CHUNK)], wsems.at[b]
        ).start()
        g = j + PREF

        @pl.when(g < NCHUNK)
        def _():
            bg = lax.rem(g, NBUF)

            @pl.when(g >= NBUF)
            def _():
                pltpu.make_async_copy(
                    buf_v.at[bg],
                    out_hbm.at[pl.ds(base + (g - NBUF) * CHUNK, CHUNK)],
                    wsems.at[bg],
                ).wait()

            gather(g, bg)

        return carry

    lax.fori_loop(0, NCHUNK, step, 0)

    for t in range(NBUF):
        j = NCHUNK - NBUF + t
        b = j % NBUF
        pltpu.make_async_copy(
            buf_v.at[b], out_hbm.at[pl.ds(base + j * CHUNK, CHUNK)], wsems.at[b]
        ).wait()


@jax.jit
def _run(targ_flat, W):
    idx = targ_flat.reshape(NW, NCHUNK, CHUNK)
    mesh = plsc.VectorSubcoreMesh(core_axis_name="c", subcore_axis_name="s")
    k = pl.kernel(
        _body,
        out_type=jax.ShapeDtypeStruct((TOTAL, EMBED_DIM), jnp.float32),
        mesh=mesh,
        scratch_types=[
            pltpu.VMEM((NCHUNK, CHUNK), jnp.int32),
            pltpu.VMEM((NBUF, CHUNK, EMBED_DIM), jnp.float32),
            pltpu.SemaphoreType.DMA((NBUF,)),
            pltpu.SemaphoreType.DMA((NBUF,)),
        ],
    )
    return k(idx, W)


def kernel(targ, W):
    out = _run(targ.reshape(TOTAL).astype(jnp.int32), W)
    return out.reshape(BATCH, HIST, EMBED_DIM)


# speedup vs baseline: 6.3214x; 1.8338x over previous
import jax
import jax.numpy as jnp
from jax import lax
from jax.experimental import pallas as pl
from jax.experimental.pallas import tpu as pltpu
from jax.experimental.pallas import tpu_sc as plsc

NODE_NUM = 100000
EMBED_DIM = 128
BATCH = 16384
HIST = 50

NC = 2
NS = 16
NW = NC * NS

B_W = BATCH // NW
KB = 2
CHUNK = KB * HIST
NCHUNK = B_W // KB

NBUF = 4
PREF = 2


def _body(idx_hbm, w_hbm, out_hbm, idx_v, buf_v, gsems, wsems):
    wid = lax.axis_index("s") * NC + lax.axis_index("c")
    b0 = wid * B_W
    pltpu.sync_copy(idx_hbm.at[wid], idx_v)

    def gather(g, bg):
        pltpu.make_async_copy(
            w_hbm.at[idx_v.at[g]], buf_v.at[bg], gsems.at[bg]
        ).start()

    def write(j, b, start):
        for i in range(KB):
            cp = pltpu.make_async_copy(
                buf_v.at[b, pl.ds(i * HIST, HIST)],
                out_hbm.at[b0 + j * KB + i],
                wsems.at[b],
            )
            cp.start() if start else cp.wait()

    for b in range(PREF):
        gather(b, b)

    def step(j, carry):
        b = lax.rem(j, NBUF)
        pltpu.make_async_copy(
            w_hbm.at[idx_v.at[j]], buf_v.at[b], gsems.at[b]
        ).wait()
        write(j, b, start=True)
        g = j + PREF

        @pl.when(g < NCHUNK)
        def _():
            bg = lax.rem(g, NBUF)

            @pl.when(g >= NBUF)
            def _():
                write(g - NBUF, bg, start=False)

            gather(g, bg)

        return carry

    lax.fori_loop(0, NCHUNK, step, 0)

    for t in range(NBUF):
        j = NCHUNK - NBUF + t
        write(j, j % NBUF, start=False)


@jax.jit
def _run(targ, W):
    idx = targ.reshape(NW, NCHUNK, CHUNK)
    mesh = plsc.VectorSubcoreMesh(core_axis_name="c", subcore_axis_name="s")
    k = pl.kernel(
        _body,
        out_type=jax.ShapeDtypeStruct((BATCH, HIST, EMBED_DIM), jnp.float32),
        mesh=mesh,
        scratch_types=[
            pltpu.VMEM((NCHUNK, CHUNK), jnp.int32),
            pltpu.VMEM((NBUF, CHUNK, EMBED_DIM), jnp.float32),
            pltpu.SemaphoreType.DMA((NBUF,)),
            pltpu.SemaphoreType.DMA((NBUF,)),
        ],
    )
    return k(idx, W)


def kernel(targ, W):
    return _run(targ.astype(jnp.int32), W)
